# baseline (speedup 1.0000x reference)
import functools

import jax
import jax.numpy as jnp
from jax import lax
from jax.experimental import pallas as pl
from jax.experimental.pallas import tpu as pltpu
from jax.experimental.pallas import tpu_sc as plsc

N_TOK = 8192
D_IN = 2048
D_OUT = 2048
N_EXP = 8

TM = 128
NT = N_TOK // TM
WMAX = NT + N_EXP - 1

_SC_NC = 2
_SC_NS = 16
_NW = _SC_NC * _SC_NS
_ROWS_PER_W = N_TOK // _NW
_CH = 16
_NB = 3
_NCH = _ROWS_PER_W // _CH


@functools.lru_cache(maxsize=None)
def _make_row_gather(n_cols):
  mesh = plsc.VectorSubcoreMesh(
      core_axis_name="c", subcore_axis_name="s",
      num_cores=_SC_NC, num_subcores=_SC_NS)

  @functools.partial(
      pl.kernel,
      out_type=jax.ShapeDtypeStruct((N_TOK, n_cols), jnp.float32),
      mesh=mesh,
      scratch_types=[
          pltpu.VMEM((_ROWS_PER_W,), jnp.int32),
          pltpu.VMEM((_NB, _CH, n_cols), jnp.float32),
          pltpu.SemaphoreType.DMA((_NB,)),
          pltpu.SemaphoreType.DMA((_NB,)),
      ],
  )
  def gather_kernel(src_hbm, idx_hbm, out_hbm, idx_v, rows_v, gsem, wsem):
    wid = lax.axis_index("s") * _SC_NC + lax.axis_index("c")
    base = wid * _ROWS_PER_W
    pltpu.sync_copy(idx_hbm.at[pl.ds(base, _ROWS_PER_W)], idx_v)

    gd = [None] * _NCH
    wd = [None] * _NCH
    for k in range(_NCH):
      b = k % _NB
      if k >= _NB:
        wd[k - _NB].wait()
      iv = idx_v[pl.ds(k * _CH, _CH)]
      gd[k] = pltpu.async_copy(src_hbm.at[iv], rows_v.at[b], gsem.at[b])
      if k >= 1:
        gd[k - 1].wait()
        wd[k - 1] = pltpu.async_copy(
            rows_v.at[(k - 1) % _NB],
            out_hbm.at[pl.ds(base + (k - 1) * _CH, _CH)],
            wsem.at[(k - 1) % _NB])
    gd[_NCH - 1].wait()
    wd[_NCH - 1] = pltpu.async_copy(
        rows_v.at[(_NCH - 1) % _NB],
        out_hbm.at[pl.ds(base + (_NCH - 1) * _CH, _CH)],
        wsem.at[(_NCH - 1) % _NB])
    for k in range(_NCH - _NB, _NCH):
      wd[k].wait()

  return gather_kernel


def _mm_body(m_ref, e_ref, lo_ref, hi_ref, xs_ref, w_ref, b_ref, out_ref):
  i = pl.program_id(0)
  lo = lo_ref[i]
  hi = hi_ref[i]

  @pl.when(hi > lo)
  def _():
    acc = jnp.dot(xs_ref[...], w_ref[0], preferred_element_type=jnp.float32)
    val = jnp.maximum(acc + b_ref[0, 0][None, :], 0.0)
    rows = lax.broadcasted_iota(jnp.int32, (TM, 1), 0)
    mask = (rows >= lo) & (rows < hi)
    out_ref[...] = jnp.where(mask, val, out_ref[...])


def _grouped_matmul(xs, w, b, m_of_w, e_of_w, lo_w, hi_w):
  grid_spec = pltpu.PrefetchScalarGridSpec(
      num_scalar_prefetch=4,
      grid=(WMAX,),
      in_specs=[
          pl.BlockSpec((TM, D_IN), lambda i, m, e, lo, hi: (m[i], 0)),
          pl.BlockSpec((1, D_IN, D_OUT), lambda i, m, e, lo, hi: (e[i], 0, 0)),
          pl.BlockSpec((1, 1, D_OUT), lambda i, m, e, lo, hi: (e[i], 0, 0)),
      ],
      out_specs=pl.BlockSpec((TM, D_OUT), lambda i, m, e, lo, hi: (m[i], 0)),
  )
  return pl.pallas_call(
      _mm_body,
      grid_spec=grid_spec,
      out_shape=jax.ShapeDtypeStruct((N_TOK, D_OUT), jnp.float32),
      compiler_params=pltpu.CompilerParams(
          dimension_semantics=("arbitrary",),
          vmem_limit_bytes=100 * 1024 * 1024,
      ),
  )(m_of_w, e_of_w, lo_w, hi_w, xs, w, b.reshape(N_EXP, 1, D_OUT))


def _schedule(e32):
  perm = jnp.argsort(e32, stable=True).astype(jnp.int32)
  pos = jnp.zeros((N_TOK,), jnp.int32).at[perm].set(
      jnp.arange(N_TOK, dtype=jnp.int32), unique_indices=True)
  counts = jnp.bincount(e32, length=N_EXP).astype(jnp.int32)
  ends = jnp.cumsum(counts)
  starts = ends - counts
  nonempty = counts > 0
  t0 = jnp.where(nonempty, starts // TM, 0)
  t1 = jnp.where(nonempty, (ends - 1) // TM, -1)
  ntiles = jnp.where(nonempty, t1 - t0 + 1, 0)
  wstart = jnp.concatenate(
      [jnp.zeros((1,), jnp.int32), jnp.cumsum(ntiles).astype(jnp.int32)])
  n_work = wstart[-1]

  wids = jnp.arange(WMAX, dtype=jnp.int32)
  valid = wids < n_work
  e_of_w = jnp.minimum(
      jnp.sum((wids[:, None] >= wstart[None, 1:]).astype(jnp.int32), axis=1),
      N_EXP - 1)
  e_last = jnp.max(jnp.where(nonempty, jnp.arange(N_EXP, dtype=jnp.int32), 0))
  e_of_w = jnp.where(valid, e_of_w, e_last)
  m_of_w = jnp.where(valid, t0[e_of_w] + (wids - wstart[e_of_w]), NT - 1)
  lo_w = jnp.where(valid, jnp.clip(starts[e_of_w] - m_of_w * TM, 0, TM), 0)
  hi_w = jnp.where(valid, jnp.clip(ends[e_of_w] - m_of_w * TM, 0, TM), 0)
  return perm, pos, m_of_w, e_of_w, lo_w.astype(jnp.int32), hi_w.astype(jnp.int32)


def kernel(x, idxs, w, b):
  e32 = idxs.astype(jnp.int32)
  perm, pos, m_of_w, e_of_w, lo_w, hi_w = _schedule(e32)
  xs = _make_row_gather(D_IN)(x, perm)
  ys = _grouped_matmul(xs, w, b, m_of_w, e_of_w, lo_w, hi_w)
  return _make_row_gather(D_OUT)(ys, pos)


# speedup vs baseline: 1.1477x; 1.1477x over previous
import functools

import jax
import jax.numpy as jnp
from jax import lax
from jax.experimental import pallas as pl
from jax.experimental.pallas import tpu as pltpu
from jax.experimental.pallas import tpu_sc as plsc

N_TOK = 8192
D_IN = 2048
D_OUT = 2048
N_EXP = 8

TM = 256
NT = N_TOK // TM
WMAX = NT + N_EXP - 1
NR = N_TOK // 128

_SC_NC = 2
_SC_NS = 16
_NW = _SC_NC * _SC_NS
_ROWS_PER_W = N_TOK // _NW
_CH = 16
_NB = 3
_NCH = _ROWS_PER_W // _CH


def _prologue_body(e_ref, pos_ref, m_ref, eo_ref, lo_ref, hi_ref):
  E = e_ref[...]
  lane = lax.broadcasted_iota(jnp.int32, (128, 128), 0)
  lane_c = lax.broadcasted_iota(jnp.int32, (128, 128), 1)
  A = (lane <= lane_c).astype(jnp.float32)
  rowi = lax.broadcasted_iota(jnp.int32, (NR, NR), 0)
  rowi_c = lax.broadcasted_iota(jnp.int32, (NR, NR), 1)
  Texc = (rowi_c < rowi).astype(jnp.float32)

  pos_acc = jnp.zeros((NR, 128), jnp.float32)
  start = jnp.float32(0.0)
  starts, ends = [], []
  for e in range(N_EXP):
    M = (E == e).astype(jnp.float32)
    P = jnp.dot(M, A, preferred_element_type=jnp.float32)
    tot = P[:, 127:128]
    B = jnp.dot(Texc, tot, preferred_element_type=jnp.float32)
    rank = B + P - 1.0
    pos_acc = pos_acc + M * (start + rank)
    starts.append(start)
    start = start + jnp.sum(M)
    ends.append(start)
  pos_ref[...] = pos_acc.astype(jnp.int32)

  wids = lax.broadcasted_iota(jnp.int32, (1, 128), 1).astype(jnp.float32)
  ws = jnp.float32(0.0)
  wstart, t0s = [], []
  for e in range(N_EXP):
    s, en = starts[e], ends[e]
    nonempty = en > s
    t0 = jnp.floor(s / TM)
    t1 = jnp.floor((en - 1.0) / TM)
    ntiles = jnp.where(nonempty, t1 - t0 + 1.0, 0.0)
    t0s.append(jnp.where(nonempty, t0, 0.0))
    wstart.append(ws)
    ws = ws + ntiles
  n_work = ws
  valid = wids < n_work
  e_of = jnp.zeros((1, 128), jnp.float32)
  for e in range(1, N_EXP):
    e_of = e_of + (wids >= wstart[e]).astype(jnp.float32)
  e_last = jnp.float32(0.0)
  for e in range(N_EXP):
    e_last = jnp.where(ends[e] > starts[e], jnp.float32(e), e_last)
  e_of = jnp.where(valid, e_of, e_last)
  t0_sel = jnp.zeros((1, 128), jnp.float32)
  ws_sel = jnp.zeros((1, 128), jnp.float32)
  st_sel = jnp.zeros((1, 128), jnp.float32)
  en_sel = jnp.zeros((1, 128), jnp.float32)
  for e in range(N_EXP):
    sel = (e_of == e).astype(jnp.float32)
    t0_sel += sel * t0s[e]
    ws_sel += sel * wstart[e]
    st_sel += sel * starts[e]
    en_sel += sel * ends[e]
  m_of = jnp.where(valid, t0_sel + (wids - ws_sel), jnp.float32(NT - 1))
  lo = jnp.where(valid, jnp.clip(st_sel - m_of * TM, 0.0, float(TM)), 0.0)
  hi = jnp.where(valid, jnp.clip(en_sel - m_of * TM, 0.0, float(TM)), 0.0)
  m_ref[...] = m_of.astype(jnp.int32)
  eo_ref[...] = e_of.astype(jnp.int32)
  lo_ref[...] = lo.astype(jnp.int32)
  hi_ref[...] = hi.astype(jnp.int32)


def _prologue(e32):
  pos, m_of, e_of, lo, hi = pl.pallas_call(
      _prologue_body,
      out_shape=[
          jax.ShapeDtypeStruct((NR, 128), jnp.int32),
          jax.ShapeDtypeStruct((1, 128), jnp.int32),
          jax.ShapeDtypeStruct((1, 128), jnp.int32),
          jax.ShapeDtypeStruct((1, 128), jnp.int32),
          jax.ShapeDtypeStruct((1, 128), jnp.int32),
      ],
  )(e32.reshape(NR, 128))
  return (pos.reshape(N_TOK), m_of.reshape(128), e_of.reshape(128),
          lo.reshape(128), hi.reshape(128))


def _sc_mesh():
  return plsc.VectorSubcoreMesh(
      core_axis_name="c", subcore_axis_name="s",
      num_cores=_SC_NC, num_subcores=_SC_NS)


@functools.lru_cache(maxsize=None)
def _make_row_scatter(n_cols):
  @functools.partial(
      pl.kernel,
      out_type=jax.ShapeDtypeStruct((N_TOK, n_cols), jnp.float32),
      mesh=_sc_mesh(),
      scratch_types=[
          pltpu.VMEM((_ROWS_PER_W,), jnp.int32),
          pltpu.VMEM((_NB, _CH, n_cols), jnp.float32),
          pltpu.SemaphoreType.DMA((_NB,)),
          pltpu.SemaphoreType.DMA((_NB,)),
      ],
  )
  def scatter_kernel(src_hbm, idx_hbm, out_hbm, idx_v, rows_v, gsem, wsem):
    wid = lax.axis_index("s") * _SC_NC + lax.axis_index("c")
    base = wid * _ROWS_PER_W
    pltpu.sync_copy(idx_hbm.at[pl.ds(base, _ROWS_PER_W)], idx_v)

    gd = [None] * _NCH
    wd = [None] * _NCH

    def put(k):
      iv = idx_v[pl.ds(k * _CH, _CH)]
      wd[k] = pltpu.async_copy(
          rows_v.at[k % _NB], out_hbm.at[iv], wsem.at[k % _NB])

    for k in range(_NCH):
      b = k % _NB
      if k >= _NB:
        wd[k - _NB].wait()
      gd[k] = pltpu.async_copy(
          src_hbm.at[pl.ds(base + k * _CH, _CH)], rows_v.at[b], gsem.at[b])
      if k >= 1:
        gd[k - 1].wait()
        put(k - 1)
    gd[_NCH - 1].wait()
    put(_NCH - 1)
    for k in range(_NCH - _NB, _NCH):
      wd[k].wait()

  return scatter_kernel


@functools.lru_cache(maxsize=None)
def _make_row_gather(n_cols):
  @functools.partial(
      pl.kernel,
      out_type=jax.ShapeDtypeStruct((N_TOK, n_cols), jnp.float32),
      mesh=_sc_mesh(),
      scratch_types=[
          pltpu.VMEM((_ROWS_PER_W,), jnp.int32),
          pltpu.VMEM((_NB, _CH, n_cols), jnp.float32),
          pltpu.SemaphoreType.DMA((_NB,)),
          pltpu.SemaphoreType.DMA((_NB,)),
      ],
  )
  def gather_kernel(src_hbm, idx_hbm, out_hbm, idx_v, rows_v, gsem, wsem):
    wid = lax.axis_index("s") * _SC_NC + lax.axis_index("c")
    base = wid * _ROWS_PER_W
    pltpu.sync_copy(idx_hbm.at[pl.ds(base, _ROWS_PER_W)], idx_v)

    gd = [None] * _NCH
    wd = [None] * _NCH

    def put(k):
      wd[k] = pltpu.async_copy(
          rows_v.at[k % _NB],
          out_hbm.at[pl.ds(base + k * _CH, _CH)],
          wsem.at[k % _NB])

    for k in range(_NCH):
      b = k % _NB
      if k >= _NB:
        wd[k - _NB].wait()
      iv = idx_v[pl.ds(k * _CH, _CH)]
      gd[k] = pltpu.async_copy(src_hbm.at[iv], rows_v.at[b], gsem.at[b])
      if k >= 1:
        gd[k - 1].wait()
        put(k - 1)
    gd[_NCH - 1].wait()
    put(_NCH - 1)
    for k in range(_NCH - _NB, _NCH):
      wd[k].wait()

  return gather_kernel


def _mm_body(m_ref, e_ref, lo_ref, hi_ref, xs_ref, w_ref, b_ref, out_ref):
  i = pl.program_id(0)
  lo = lo_ref[i]
  hi = hi_ref[i]

  @pl.when(hi > lo)
  def _():
    acc = jnp.dot(xs_ref[...], w_ref[0], preferred_element_type=jnp.float32)
    val = jnp.maximum(acc + b_ref[0, 0][None, :], 0.0)
    rows = lax.broadcasted_iota(jnp.int32, (TM, 1), 0)
    mask = (rows >= lo) & (rows < hi)
    out_ref[...] = jnp.where(mask, val, out_ref[...])


def _grouped_matmul(xs, w, b, m_of_w, e_of_w, lo_w, hi_w):
  grid_spec = pltpu.PrefetchScalarGridSpec(
      num_scalar_prefetch=4,
      grid=(WMAX,),
      in_specs=[
          pl.BlockSpec((TM, D_IN), lambda i, m, e, lo, hi: (m[i], 0)),
          pl.BlockSpec((1, D_IN, D_OUT), lambda i, m, e, lo, hi: (e[i], 0, 0)),
          pl.BlockSpec((1, 1, D_OUT), lambda i, m, e, lo, hi: (e[i], 0, 0)),
      ],
      out_specs=pl.BlockSpec((TM, D_OUT), lambda i, m, e, lo, hi: (m[i], 0)),
  )
  return pl.pallas_call(
      _mm_body,
      grid_spec=grid_spec,
      out_shape=jax.ShapeDtypeStruct((N_TOK, D_OUT), jnp.float32),
      compiler_params=pltpu.CompilerParams(
          dimension_semantics=("arbitrary",),
          vmem_limit_bytes=100 * 1024 * 1024,
      ),
  )(m_of_w, e_of_w, lo_w, hi_w, xs, w, b.reshape(N_EXP, 1, D_OUT))


def kernel(x, idxs, w, b):
  e32 = idxs.astype(jnp.int32)
  pos, m_of_w, e_of_w, lo_w, hi_w = _prologue(e32)
  xs = _make_row_scatter(D_IN)(x, pos)
  ys = _grouped_matmul(xs, w, b, m_of_w, e_of_w, lo_w, hi_w)
  return _make_row_gather(D_OUT)(ys, pos)
